# baseline (speedup 1.0000x reference)
import functools

import jax
import jax.numpy as jnp
from jax import lax
from jax.experimental import pallas as pl
from jax.experimental.pallas import tpu as pltpu
from jax.experimental.pallas import tpu_sc as plsc

_B = 16
_D = 1024
_NS = 2
_BLK = 2048
_NW = 32
_CH = 32


def _eye(n, dtype):
    return (jax.lax.broadcasted_iota(jnp.int32, (n, n), 0)
            == jax.lax.broadcasted_iota(jnp.int32, (n, n), 1)).astype(dtype)


def _vgather(x, idx):
    dnums = lax.GatherDimensionNumbers(
        offset_dims=(), collapsed_slice_dims=(0,), start_index_map=(0,))
    return lax.gather(x, idx[:, None], dnums, slice_sizes=(1,),
                      mode=lax.GatherScatterMode.PROMISE_IN_BOUNDS)


def _tc_body(*refs):
    ids_ref = refs[0]
    x_refs = refs[1:1 + _NS]
    att_ref = refs[1 + _NS]
    out_refs = refs[2 + _NS:2 + 2 * _NS]
    m_out_ref = refs[2 + 2 * _NS]
    s_out_ref = refs[3 + 2 * _NS]
    m_ref, s_ref = refs[4 + 2 * _NS:]

    i = pl.program_id(0)
    nb = pl.num_programs(0)

    @pl.when(i == 0)
    def _init():
        m_ref[...] = jnp.full_like(m_ref, -jnp.inf)
        s_ref[...] = jnp.zeros_like(s_ref)
        for o in out_refs:
            o[...] = jnp.zeros_like(o)

    xs = [r[...].astype(jnp.bfloat16) for r in x_refs]
    att = att_ref[...].astype(jnp.bfloat16)
    hd = xs[0].shape[1]
    dn = (((1,), (0,)), ((), ()))
    l = sum(jax.lax.dot_general(x, att[j * hd:(j + 1) * hd], dn,
                                preferred_element_type=jnp.float32)
            for j, x in enumerate(xs))
    ids = ids_ref[...]
    oh = ids == jax.lax.broadcasted_iota(jnp.int32, (1, _B), 1)

    m_old = m_ref[...]
    bm = jnp.max(jnp.where(oh, l, -jnp.inf), axis=0, keepdims=True)
    m_new = jnp.maximum(m_old, bm)
    scale = jnp.where(m_old == -jnp.inf, 0.0, jnp.exp(m_old - m_new))
    p = jnp.exp(jnp.where(oh, l - m_new, -jnp.inf))

    s_ref[...] = s_ref[...] * scale + jnp.sum(p, axis=0, keepdims=True)
    m_ref[...] = m_new

    eye = _eye(_B, jnp.float32)
    tdn = (((1,), (1,)), ((), ()))
    scale_col = jax.lax.dot_general(eye, scale, tdn,
                                    preferred_element_type=jnp.float32)
    ph = p.astype(jnp.bfloat16)
    cdn = (((0,), (0,)), ((), ()))
    for o, x in zip(out_refs, xs):
        o[...] = o[...] * scale_col + jax.lax.dot_general(
            ph, x, cdn, preferred_element_type=jnp.float32)

    @pl.when(i == nb - 1)
    def _fin():
        m_out_ref[...] = m_ref[...]
        s_out_ref[...] = s_ref[...]


def _tc_run(ids, flat, att2, r):
    n, d = flat.shape
    hd = d // _NS

    def xspec(j):
        return pl.BlockSpec((_BLK, hd), lambda i, j=j: (i, j))

    return pl.pallas_call(
        _tc_body,
        grid=(r // _BLK,),
        in_specs=(
            [pl.BlockSpec((_BLK, 1), lambda i: (i, 0))]
            + [xspec(j) for j in range(_NS)]
            + [pl.BlockSpec((d, 1), lambda i: (0, 0))]
        ),
        out_specs=(
            [pl.BlockSpec((_B, hd), lambda i: (0, 0))] * _NS
            + [pl.BlockSpec((1, _B), lambda i: (0, 0))] * 2
        ),
        out_shape=(
            [jax.ShapeDtypeStruct((_B, hd), jnp.float32)] * _NS
            + [jax.ShapeDtypeStruct((1, _B), jnp.float32)] * 2
        ),
        scratch_shapes=[
            pltpu.VMEM((1, _B), jnp.float32),
            pltpu.VMEM((1, _B), jnp.float32),
        ],
    )(ids, *([flat] * _NS), att2)


def _sc_body(r0, rpw, flat_hbm, ids_hbm, att_hbm, acc_hbm, s_hbm,
             buf0, buf1, acc_v, att_v, ids_all, pbuf, s_v, sem0, sem1):
    cid = lax.axis_index("c")
    sid = lax.axis_index("s")
    wid = sid * 2 + cid
    base = r0 + wid * rpw
    nch = rpw // _CH
    lane = lax.iota(jnp.int32, 16)

    pltpu.sync_copy(att_hbm, att_v)
    pltpu.sync_copy(ids_hbm.at[pl.ds(base, rpw)], ids_all)

    @plsc.parallel_loop(0, _B * _D // 16, unroll=16)
    def _zj(j):
        acc_v[j // (_D // 16), pl.ds((j % (_D // 16)) * 16, 16)] = jnp.zeros(
            (16,), jnp.float32)

    bufs = (buf0, buf1)
    sems = (sem0, sem1)
    descs = [None, None]
    descs[0] = pltpu.async_copy(
        flat_hbm.at[pl.ds(base, _CH)], bufs[0], sems[0])
    s_vec = jnp.zeros((16,), jnp.float32)

    for c in range(nch):
        if c + 1 < nch:
            descs[(c + 1) % 2] = pltpu.async_copy(
                flat_hbm.at[pl.ds(base + (c + 1) * _CH, _CH)],
                bufs[(c + 1) % 2], sems[(c + 1) % 2])
        descs[c % 2].wait()
        buf = bufs[c % 2]

        for g in range(_CH // 16):
            gvec = ids_all[pl.ds(c * _CH + g * 16, 16)]
            rid0 = _vgather(gvec, jnp.zeros((16,), jnp.int32))[0]
            rid15 = _vgather(gvec, jnp.full((16,), 15, jnp.int32))[0]

            def _row_w(r, buf=buf):
                z = jnp.zeros((16,), jnp.float32)

                @plsc.parallel_loop(0, _D // 64, unroll=4, carry=(z, z, z, z))
                def p4(j4, ps):
                    return tuple(
                        ps[u] + buf[r, pl.ds((j4 * 4 + u) * 16, 16)]
                        * att_v[pl.ds((j4 * 4 + u) * 16, 16)]
                        for u in range(4))
                part = (p4[0] + p4[1]) + (p4[2] + p4[3])
                for k in (8, 4, 2, 1):
                    part = part + _vgather(part, lane ^ k)
                return jnp.exp(part)

            def _fast(s_acc, buf=buf, g=g, rid=rid0):
                def _wst(r, t):
                    pbuf[r, :] = _row_w(g * 16 + r, buf=buf)
                    return t
                lax.fori_loop(0, 16, _wst, 0)
                wl = [pbuf[u, :] for u in range(16)]

                @plsc.parallel_loop(0, _D // 16, unroll=4)
                def _updg(j):
                    sl = pl.ds(j * 16, 16)
                    t0 = acc_v[rid, sl]
                    t1 = wl[1] * buf[g * 16 + 1, sl]
                    for u in range(0, 16, 2):
                        t0 = t0 + wl[u] * buf[g * 16 + u, sl]
                    for u in range(3, 16, 2):
                        t1 = t1 + wl[u] * buf[g * 16 + u, sl]
                    acc_v[rid, sl] = t0 + t1
                wsum = sum(wl[1:], wl[0])
                return s_acc + jnp.where(lane == rid, wsum, 0.0)

            def _slow(s_acc, buf=buf, g=g, gvec=gvec):
                def _row(r, s_acc):
                    w = _row_w(g * 16 + r, buf=buf)
                    rid = _vgather(gvec, jnp.full((16,), r, jnp.int32))[0]

                    @plsc.parallel_loop(0, _D // 16, unroll=16)
                    def _upd(j):
                        sl = pl.ds(j * 16, 16)
                        acc_v[rid, sl] = (acc_v[rid, sl]
                                          + w * buf[g * 16 + r, sl])
                    return s_acc + jnp.where(lane == rid, w, 0.0)
                return lax.fori_loop(0, 16, _row, s_acc)

            s_vec = lax.cond(rid0 == rid15, _fast, _slow, s_vec)

    s_v[...] = s_vec
    pltpu.sync_copy(acc_v, acc_hbm.at[wid])
    pltpu.sync_copy(s_v, s_hbm.at[wid])


def _sc_run(flat, ids1d, att1d, r0, rpw):
    body = functools.partial(_sc_body, r0, rpw)
    return pl.kernel(
        body,
        out_type=[
            jax.ShapeDtypeStruct((_NW, _B, _D), jnp.float32),
            jax.ShapeDtypeStruct((_NW, _B), jnp.float32),
        ],
        mesh=plsc.VectorSubcoreMesh(core_axis_name="c", subcore_axis_name="s"),
        compiler_params=pltpu.CompilerParams(needs_layout_passes=False),
        scratch_types=[
            pltpu.VMEM((_CH, _D), jnp.float32),
            pltpu.VMEM((_CH, _D), jnp.float32),
            pltpu.VMEM((_B, _D), jnp.float32),
            pltpu.VMEM((_D,), jnp.float32),
            pltpu.VMEM((rpw,), jnp.int32),
            pltpu.VMEM((16, 16), jnp.float32),
            pltpu.VMEM((_B,), jnp.float32),
            pltpu.SemaphoreType.DMA,
            pltpu.SemaphoreType.DMA,
        ],
    )(flat, ids1d, att1d)


def _merge_body(t1_ref, t2_ref, m_ref, s_ref, acc_sc_ref, s_sc_ref, out_ref):
    asum = jnp.sum(acc_sc_ref[...], axis=0)
    ssum = jnp.sum(s_sc_ref[...], axis=0, keepdims=True)
    m_tc = m_ref[...]
    mstar = jnp.maximum(m_tc, 0.0)
    f_tc = jnp.exp(m_tc - mstar)
    f_sc = jnp.exp(-mstar)
    den = s_ref[...] * f_tc + ssum * f_sc
    inv = jnp.where(den > 0.0, 1.0 / den, 0.0)
    eye = _eye(_B, jnp.float32)
    tdn = (((1,), (1,)), ((), ()))
    c_tc = jax.lax.dot_general(eye, f_tc * inv, tdn,
                               preferred_element_type=jnp.float32)
    c_sc = jax.lax.dot_general(eye, f_sc * inv, tdn,
                               preferred_element_type=jnp.float32)
    hd = _D // _NS
    acc_tc = jnp.concatenate([t1_ref[...], t2_ref[...]], axis=1)
    out_ref[...] = acc_tc * c_tc + asum * c_sc


def _merge(t1, t2, m_tc, s_tc, acc_sc, s_sc):
    return pl.pallas_call(
        _merge_body,
        out_shape=jax.ShapeDtypeStruct((_B, _D), jnp.float32),
    )(t1, t2, m_tc, s_tc, acc_sc, s_sc)


@jax.jit
def kernel(flat, segment_ids, att, bias, temperature):
    n, d = flat.shape
    r = (3 * n) // 4
    att2 = att * temperature[0]
    ids = segment_ids.astype(jnp.int32)
    acc_sc, s_sc = _sc_run(flat, ids, att2.reshape(d), r, (n - r) // _NW)
    o = _tc_run(ids.reshape(n, 1), flat, att2, r)
    return _merge(o[0], o[1], o[_NS], o[_NS + 1], acc_sc, s_sc)


# speedup vs baseline: 1.0311x; 1.0311x over previous
import functools

import jax
import jax.numpy as jnp
from jax import lax
from jax.experimental import pallas as pl
from jax.experimental.pallas import tpu as pltpu
from jax.experimental.pallas import tpu_sc as plsc

_B = 16
_D = 1024
_NS = 2
_BLK = 2048
_NW = 32
_CH = 32


def _eye(n, dtype):
    return (jax.lax.broadcasted_iota(jnp.int32, (n, n), 0)
            == jax.lax.broadcasted_iota(jnp.int32, (n, n), 1)).astype(dtype)


def _vgather(x, idx):
    dnums = lax.GatherDimensionNumbers(
        offset_dims=(), collapsed_slice_dims=(0,), start_index_map=(0,))
    return lax.gather(x, idx[:, None], dnums, slice_sizes=(1,),
                      mode=lax.GatherScatterMode.PROMISE_IN_BOUNDS)


def _tc_body(*refs):
    ids_ref = refs[0]
    x_refs = refs[1:1 + _NS]
    att_ref = refs[1 + _NS]
    out_refs = refs[2 + _NS:2 + 2 * _NS]
    m_out_ref = refs[2 + 2 * _NS]
    s_out_ref = refs[3 + 2 * _NS]
    m_ref, s_ref = refs[4 + 2 * _NS:]

    i = pl.program_id(0)
    nb = pl.num_programs(0)

    @pl.when(i == 0)
    def _init():
        m_ref[...] = jnp.full_like(m_ref, -jnp.inf)
        s_ref[...] = jnp.zeros_like(s_ref)
        for o in out_refs:
            o[...] = jnp.zeros_like(o)

    xs = [r[...].astype(jnp.bfloat16) for r in x_refs]
    att = att_ref[...].astype(jnp.bfloat16)
    hd = xs[0].shape[1]
    dn = (((1,), (0,)), ((), ()))
    l = sum(jax.lax.dot_general(x, att[j * hd:(j + 1) * hd], dn,
                                preferred_element_type=jnp.float32)
            for j, x in enumerate(xs))
    ids = ids_ref[...]
    oh = ids == jax.lax.broadcasted_iota(jnp.int32, (1, _B), 1)

    m_old = m_ref[...]
    bm = jnp.max(jnp.where(oh, l, -jnp.inf), axis=0, keepdims=True)
    m_new = jnp.maximum(m_old, bm)
    scale = jnp.where(m_old == -jnp.inf, 0.0, jnp.exp(m_old - m_new))
    p = jnp.exp(jnp.where(oh, l - m_new, -jnp.inf))

    s_ref[...] = s_ref[...] * scale + jnp.sum(p, axis=0, keepdims=True)
    m_ref[...] = m_new

    eye = _eye(_B, jnp.float32)
    tdn = (((1,), (1,)), ((), ()))
    scale_col = jax.lax.dot_general(eye, scale, tdn,
                                    preferred_element_type=jnp.float32)
    ph = p.astype(jnp.bfloat16)
    cdn = (((0,), (0,)), ((), ()))
    for o, x in zip(out_refs, xs):
        o[...] = o[...] * scale_col + jax.lax.dot_general(
            ph, x, cdn, preferred_element_type=jnp.float32)

    @pl.when(i == nb - 1)
    def _fin():
        m_out_ref[...] = m_ref[...]
        s_out_ref[...] = s_ref[...]


def _tc_run(ids, flat, att2, r):
    n, d = flat.shape
    hd = d // _NS

    def xspec(j):
        return pl.BlockSpec((_BLK, hd), lambda i, j=j: (i, j))

    return pl.pallas_call(
        _tc_body,
        grid=(r // _BLK,),
        in_specs=(
            [pl.BlockSpec((_BLK, 1), lambda i: (i, 0))]
            + [xspec(j) for j in range(_NS)]
            + [pl.BlockSpec((d, 1), lambda i: (0, 0))]
        ),
        out_specs=(
            [pl.BlockSpec((_B, hd), lambda i: (0, 0))] * _NS
            + [pl.BlockSpec((1, _B), lambda i: (0, 0))] * 2
        ),
        out_shape=(
            [jax.ShapeDtypeStruct((_B, hd), jnp.float32)] * _NS
            + [jax.ShapeDtypeStruct((1, _B), jnp.float32)] * 2
        ),
        scratch_shapes=[
            pltpu.VMEM((1, _B), jnp.float32),
            pltpu.VMEM((1, _B), jnp.float32),
        ],
    )(ids, *([flat] * _NS), att2)


def _sc_body(r0, rpw, flat_hbm, ids_hbm, att_hbm, acc_hbm, s_hbm,
             buf0, buf1, acc_v, att_v, ids_all, s_v, sem0, sem1):
    cid = lax.axis_index("c")
    sid = lax.axis_index("s")
    wid = sid * 2 + cid
    base = r0 + wid * rpw
    nch = rpw // _CH
    lane = lax.iota(jnp.int32, 16)

    pltpu.sync_copy(att_hbm, att_v)
    pltpu.sync_copy(ids_hbm.at[pl.ds(base, rpw)], ids_all)

    @plsc.parallel_loop(0, _B * _D // 16, unroll=16)
    def _zj(j):
        acc_v[j // (_D // 16), pl.ds((j % (_D // 16)) * 16, 16)] = jnp.zeros(
            (16,), jnp.float32)

    bufs = (buf0, buf1)
    sems = (sem0, sem1)
    descs = [None, None]
    descs[0] = pltpu.async_copy(
        flat_hbm.at[pl.ds(base, _CH)], bufs[0], sems[0])
    s_vec = jnp.zeros((16,), jnp.float32)

    for c in range(nch):
        if c + 1 < nch:
            descs[(c + 1) % 2] = pltpu.async_copy(
                flat_hbm.at[pl.ds(base + (c + 1) * _CH, _CH)],
                bufs[(c + 1) % 2], sems[(c + 1) % 2])
        descs[c % 2].wait()
        buf = bufs[c % 2]

        for g in range(_CH // 16):
            gvec = ids_all[pl.ds(c * _CH + g * 16, 16)]

            def _row(r, s_acc, buf=buf, gvec=gvec, g=g):
                rid = _vgather(gvec, jnp.full((16,), r, jnp.int32))[0]

                z = jnp.zeros((16,), jnp.float32)

                @plsc.parallel_loop(0, _D // 64, unroll=4, carry=(z, z, z, z))
                def p4(j4, ps):
                    return tuple(
                        ps[u] + buf[g * 16 + r, pl.ds((j4 * 4 + u) * 16, 16)]
                        * att_v[pl.ds((j4 * 4 + u) * 16, 16)]
                        for u in range(4))
                part = (p4[0] + p4[1]) + (p4[2] + p4[3])
                for k in (8, 4, 2, 1):
                    part = part + _vgather(part, lane ^ k)
                w = jnp.exp(part)

                @plsc.parallel_loop(0, _D // 16, unroll=16)
                def _upd(j):
                    sl = pl.ds(j * 16, 16)
                    acc_v[rid, sl] = acc_v[rid, sl] + w * buf[g * 16 + r, sl]
                return s_acc + jnp.where(lane == rid, w, 0.0)

            s_vec = lax.fori_loop(0, 16, _row, s_vec)

    s_v[...] = s_vec
    pltpu.sync_copy(acc_v, acc_hbm.at[wid])
    pltpu.sync_copy(s_v, s_hbm.at[wid])


def _sc_run(flat, ids1d, att1d, r0, rpw):
    body = functools.partial(_sc_body, r0, rpw)
    return pl.kernel(
        body,
        out_type=[
            jax.ShapeDtypeStruct((_NW, _B, _D), jnp.float32),
            jax.ShapeDtypeStruct((_NW, _B), jnp.float32),
        ],
        mesh=plsc.VectorSubcoreMesh(core_axis_name="c", subcore_axis_name="s"),
        compiler_params=pltpu.CompilerParams(needs_layout_passes=False),
        scratch_types=[
            pltpu.VMEM((_CH, _D), jnp.float32),
            pltpu.VMEM((_CH, _D), jnp.float32),
            pltpu.VMEM((_B, _D), jnp.float32),
            pltpu.VMEM((_D,), jnp.float32),
            pltpu.VMEM((rpw,), jnp.int32),
            pltpu.VMEM((_B,), jnp.float32),
            pltpu.SemaphoreType.DMA,
            pltpu.SemaphoreType.DMA,
        ],
    )(flat, ids1d, att1d)


def _merge_body(t1_ref, t2_ref, m_ref, s_ref, acc_sc_ref, s_sc_ref, out_ref):
    asum = jnp.sum(acc_sc_ref[...], axis=0)
    ssum = jnp.sum(s_sc_ref[...], axis=0, keepdims=True)
    m_tc = m_ref[...]
    mstar = jnp.maximum(m_tc, 0.0)
    f_tc = jnp.exp(m_tc - mstar)
    f_sc = jnp.exp(-mstar)
    den = s_ref[...] * f_tc + ssum * f_sc
    inv = jnp.where(den > 0.0, 1.0 / den, 0.0)
    eye = _eye(_B, jnp.float32)
    tdn = (((1,), (1,)), ((), ()))
    c_tc = jax.lax.dot_general(eye, f_tc * inv, tdn,
                               preferred_element_type=jnp.float32)
    c_sc = jax.lax.dot_general(eye, f_sc * inv, tdn,
                               preferred_element_type=jnp.float32)
    hd = _D // _NS
    acc_tc = jnp.concatenate([t1_ref[...], t2_ref[...]], axis=1)
    out_ref[...] = acc_tc * c_tc + asum * c_sc


def _merge(t1, t2, m_tc, s_tc, acc_sc, s_sc):
    return pl.pallas_call(
        _merge_body,
        out_shape=jax.ShapeDtypeStruct((_B, _D), jnp.float32),
    )(t1, t2, m_tc, s_tc, acc_sc, s_sc)


@jax.jit
def kernel(flat, segment_ids, att, bias, temperature):
    n, d = flat.shape
    r = (3 * n) // 4
    att2 = att * temperature[0]
    ids = segment_ids.astype(jnp.int32)
    acc_sc, s_sc = _sc_run(flat, ids, att2.reshape(d), r, (n - r) // _NW)
    o = _tc_run(ids.reshape(n, 1), flat, att2, r)
    return _merge(o[0], o[1], o[_NS], o[_NS + 1], acc_sc, s_sc)


# speedup vs baseline: 1.0368x; 1.0055x over previous
import functools

import jax
import jax.numpy as jnp
from jax import lax
from jax.experimental import pallas as pl
from jax.experimental.pallas import tpu as pltpu
from jax.experimental.pallas import tpu_sc as plsc

_B = 16
_D = 1024
_NS = 2
_BLK = 1536
_NW = 32
_CH = 32


def _eye(n, dtype):
    return (jax.lax.broadcasted_iota(jnp.int32, (n, n), 0)
            == jax.lax.broadcasted_iota(jnp.int32, (n, n), 1)).astype(dtype)


def _vgather(x, idx):
    dnums = lax.GatherDimensionNumbers(
        offset_dims=(), collapsed_slice_dims=(0,), start_index_map=(0,))
    return lax.gather(x, idx[:, None], dnums, slice_sizes=(1,),
                      mode=lax.GatherScatterMode.PROMISE_IN_BOUNDS)


def _tc_body(*refs):
    ids_ref = refs[0]
    x_refs = refs[1:1 + _NS]
    att_ref = refs[1 + _NS]
    out_refs = refs[2 + _NS:2 + 2 * _NS]
    m_out_ref = refs[2 + 2 * _NS]
    s_out_ref = refs[3 + 2 * _NS]
    m_ref, s_ref = refs[4 + 2 * _NS:]

    i = pl.program_id(0)
    nb = pl.num_programs(0)

    @pl.when(i == 0)
    def _init():
        m_ref[...] = jnp.full_like(m_ref, -jnp.inf)
        s_ref[...] = jnp.zeros_like(s_ref)
        for o in out_refs:
            o[...] = jnp.zeros_like(o)

    xs = [r[...].astype(jnp.bfloat16) for r in x_refs]
    att = att_ref[...].astype(jnp.bfloat16)
    hd = xs[0].shape[1]
    dn = (((1,), (0,)), ((), ()))
    l = sum(jax.lax.dot_general(x, att[j * hd:(j + 1) * hd], dn,
                                preferred_element_type=jnp.float32)
            for j, x in enumerate(xs))
    ids = ids_ref[...]
    oh = ids == jax.lax.broadcasted_iota(jnp.int32, (1, _B), 1)

    m_old = m_ref[...]
    bm = jnp.max(jnp.where(oh, l, -jnp.inf), axis=0, keepdims=True)
    m_new = jnp.maximum(m_old, bm)
    scale = jnp.where(m_old == -jnp.inf, 0.0, jnp.exp(m_old - m_new))
    p = jnp.exp(jnp.where(oh, l - m_new, -jnp.inf))

    s_ref[...] = s_ref[...] * scale + jnp.sum(p, axis=0, keepdims=True)
    m_ref[...] = m_new

    eye = _eye(_B, jnp.float32)
    tdn = (((1,), (1,)), ((), ()))
    scale_col = jax.lax.dot_general(eye, scale, tdn,
                                    preferred_element_type=jnp.float32)
    ph = p.astype(jnp.bfloat16)
    cdn = (((0,), (0,)), ((), ()))
    for o, x in zip(out_refs, xs):
        o[...] = o[...] * scale_col + jax.lax.dot_general(
            ph, x, cdn, preferred_element_type=jnp.float32)

    @pl.when(i == nb - 1)
    def _fin():
        m_out_ref[...] = m_ref[...]
        s_out_ref[...] = s_ref[...]


def _tc_run(ids, flat, att2, r):
    n, d = flat.shape
    hd = d // _NS

    def xspec(j):
        return pl.BlockSpec((_BLK, hd), lambda i, j=j: (i, j))

    return pl.pallas_call(
        _tc_body,
        grid=(r // _BLK,),
        in_specs=(
            [pl.BlockSpec((_BLK, 1), lambda i: (i, 0))]
            + [xspec(j) for j in range(_NS)]
            + [pl.BlockSpec((d, 1), lambda i: (0, 0))]
        ),
        out_specs=(
            [pl.BlockSpec((_B, hd), lambda i: (0, 0))] * _NS
            + [pl.BlockSpec((1, _B), lambda i: (0, 0))] * 2
        ),
        out_shape=(
            [jax.ShapeDtypeStruct((_B, hd), jnp.float32)] * _NS
            + [jax.ShapeDtypeStruct((1, _B), jnp.float32)] * 2
        ),
        scratch_shapes=[
            pltpu.VMEM((1, _B), jnp.float32),
            pltpu.VMEM((1, _B), jnp.float32),
        ],
    )(ids, *([flat] * _NS), att2)


def _sc_body(r0, rpw, flat_hbm, ids_hbm, att_hbm, acc_hbm, s_hbm,
             buf0, buf1, acc_v, att_v, ids_all, s_v, sem0, sem1):
    cid = lax.axis_index("c")
    sid = lax.axis_index("s")
    wid = sid * 2 + cid
    base = r0 + wid * rpw
    nch = rpw // _CH
    lane = lax.iota(jnp.int32, 16)

    pltpu.sync_copy(att_hbm, att_v)
    pltpu.sync_copy(ids_hbm.at[pl.ds(base, rpw)], ids_all)

    @plsc.parallel_loop(0, _B * _D // 16, unroll=16)
    def _zj(j):
        acc_v[j // (_D // 16), pl.ds((j % (_D // 16)) * 16, 16)] = jnp.zeros(
            (16,), jnp.float32)

    bufs = (buf0, buf1)
    sems = (sem0, sem1)
    descs = [None, None]
    descs[0] = pltpu.async_copy(
        flat_hbm.at[pl.ds(base, _CH)], bufs[0], sems[0])
    s_vec = jnp.zeros((16,), jnp.float32)

    for c in range(nch):
        if c + 1 < nch:
            descs[(c + 1) % 2] = pltpu.async_copy(
                flat_hbm.at[pl.ds(base + (c + 1) * _CH, _CH)],
                bufs[(c + 1) % 2], sems[(c + 1) % 2])
        descs[c % 2].wait()
        buf = bufs[c % 2]

        for g in range(_CH // 16):
            gvec = ids_all[pl.ds(c * _CH + g * 16, 16)]

            def _row(r, s_acc, buf=buf, gvec=gvec, g=g):
                rid = _vgather(gvec, jnp.full((16,), r, jnp.int32))[0]

                z = jnp.zeros((16,), jnp.float32)

                @plsc.parallel_loop(0, _D // 64, unroll=4, carry=(z, z, z, z))
                def p4(j4, ps):
                    return tuple(
                        ps[u] + buf[g * 16 + r, pl.ds((j4 * 4 + u) * 16, 16)]
                        * att_v[pl.ds((j4 * 4 + u) * 16, 16)]
                        for u in range(4))
                part = (p4[0] + p4[1]) + (p4[2] + p4[3])
                for k in (8, 4, 2, 1):
                    part = part + _vgather(part, lane ^ k)
                w = jnp.exp(part)

                @plsc.parallel_loop(0, _D // 16, unroll=16)
                def _upd(j):
                    sl = pl.ds(j * 16, 16)
                    acc_v[rid, sl] = acc_v[rid, sl] + w * buf[g * 16 + r, sl]
                return s_acc + jnp.where(lane == rid, w, 0.0)

            s_vec = lax.fori_loop(0, 16, _row, s_vec)

    s_v[...] = s_vec
    pltpu.sync_copy(acc_v, acc_hbm.at[wid])
    pltpu.sync_copy(s_v, s_hbm.at[wid])


def _sc_run(flat, ids1d, att1d, r0, rpw):
    body = functools.partial(_sc_body, r0, rpw)
    return pl.kernel(
        body,
        out_type=[
            jax.ShapeDtypeStruct((_NW, _B, _D), jnp.float32),
            jax.ShapeDtypeStruct((_NW, _B), jnp.float32),
        ],
        mesh=plsc.VectorSubcoreMesh(core_axis_name="c", subcore_axis_name="s"),
        compiler_params=pltpu.CompilerParams(needs_layout_passes=False),
        scratch_types=[
            pltpu.VMEM((_CH, _D), jnp.float32),
            pltpu.VMEM((_CH, _D), jnp.float32),
            pltpu.VMEM((_B, _D), jnp.float32),
            pltpu.VMEM((_D,), jnp.float32),
            pltpu.VMEM((rpw,), jnp.int32),
            pltpu.VMEM((_B,), jnp.float32),
            pltpu.SemaphoreType.DMA,
            pltpu.SemaphoreType.DMA,
        ],
    )(flat, ids1d, att1d)


def _merge_body(t1_ref, t2_ref, m_ref, s_ref, acc_sc_ref, s_sc_ref, out_ref):
    asum = jnp.sum(acc_sc_ref[...], axis=0)
    ssum = jnp.sum(s_sc_ref[...], axis=0, keepdims=True)
    m_tc = m_ref[...]
    mstar = jnp.maximum(m_tc, 0.0)
    f_tc = jnp.exp(m_tc - mstar)
    f_sc = jnp.exp(-mstar)
    den = s_ref[...] * f_tc + ssum * f_sc
    inv = jnp.where(den > 0.0, 1.0 / den, 0.0)
    eye = _eye(_B, jnp.float32)
    tdn = (((1,), (1,)), ((), ()))
    c_tc = jax.lax.dot_general(eye, f_tc * inv, tdn,
                               preferred_element_type=jnp.float32)
    c_sc = jax.lax.dot_general(eye, f_sc * inv, tdn,
                               preferred_element_type=jnp.float32)
    hd = _D // _NS
    acc_tc = jnp.concatenate([t1_ref[...], t2_ref[...]], axis=1)
    out_ref[...] = acc_tc * c_tc + asum * c_sc


def _merge(t1, t2, m_tc, s_tc, acc_sc, s_sc):
    return pl.pallas_call(
        _merge_body,
        out_shape=jax.ShapeDtypeStruct((_B, _D), jnp.float32),
    )(t1, t2, m_tc, s_tc, acc_sc, s_sc)


@jax.jit
def kernel(flat, segment_ids, att, bias, temperature):
    n, d = flat.shape
    r = (3 * n) // 4
    att2 = att * temperature[0]
    ids = segment_ids.astype(jnp.int32)
    acc_sc, s_sc = _sc_run(flat, ids, att2.reshape(d), r, (n - r) // _NW)
    o = _tc_run(ids.reshape(n, 1), flat, att2, r)
    return _merge(o[0], o[1], o[_NS], o[_NS + 1], acc_sc, s_sc)
